# baseline (speedup 1.0000x reference)
import jax
import jax.numpy as jnp
from jax.experimental import pallas as pl
from jax.experimental.pallas import tpu as pltpu

_B, _S, _E = 4, 2048, 4096
_P = 10
_R = _S // 8
_CH = 32
_NBUF = 4
_NCH = _R // _CH


def _copy(x_hbm, bufs, sems, c):
    bi, k = divmod(c, _NCH)
    return pltpu.make_async_copy(
        x_hbm.at[bi, pl.ds(k * _CH, _CH)],
        bufs.at[c % _NBUF],
        sems.at[c % _NBUF],
    )


def _body(x_hbm, w_ref, b_ref, p_hbm, logits_ref, ep_ref,
          accs, bufs, sems, gsems):
    total = _B * _NCH
    for c in range(_NBUF):
        _copy(x_hbm, bufs, sems, c).start()
    acc = None
    for c in range(total):
        bi, k = divmod(c, _NCH)
        _copy(x_hbm, bufs, sems, c).wait()
        part = jnp.max(bufs[c % _NBUF], axis=0)
        acc = part if k == 0 else jnp.maximum(acc, part)
        if c + _NBUF < total:
            _copy(x_hbm, bufs, sems, c + _NBUF).start()
        if k == _NCH - 1:
            accs[pl.ds(bi, 1), :] = jnp.max(acc, axis=0, keepdims=True)
    logits = jax.lax.dot_general(
        accs[...], w_ref[...],
        (((1,), (1,)), ((), ())),
        preferred_element_type=jnp.float32,
    ) + b_ref[...]
    logits_ref[...] = logits
    iota = jax.lax.broadcasted_iota(jnp.int32, (_B, _P), 1)
    m = jnp.max(logits, axis=1, keepdims=True)
    idxv = jnp.min(jnp.where(logits == m, iota, _P), axis=1, keepdims=True)
    gathers = []
    for bi in range(_B):
        rid = idxv[bi, 0]
        for t in range(2):
            g = pltpu.make_async_copy(
                p_hbm.at[rid + t * _P],
                ep_ref.at[2 * bi + t],
                gsems.at[2 * bi + t],
            )
            g.start()
            gathers.append(g)
    for g in gathers:
        g.wait()


def kernel(x_embed, prompt, W, b):
    x4 = x_embed.reshape(_B, _R, 8, _E)
    p3 = prompt.reshape(2 * _P, 160, 128)
    b2 = b.reshape(1, _P)
    logits, ep = pl.pallas_call(
        _body,
        in_specs=[
            pl.BlockSpec(memory_space=pltpu.MemorySpace.HBM),
            pl.BlockSpec(memory_space=pltpu.MemorySpace.VMEM),
            pl.BlockSpec(memory_space=pltpu.MemorySpace.VMEM),
            pl.BlockSpec(memory_space=pltpu.MemorySpace.HBM),
        ],
        out_specs=[
            pl.BlockSpec(memory_space=pltpu.MemorySpace.VMEM),
            pl.BlockSpec(memory_space=pltpu.MemorySpace.VMEM),
        ],
        out_shape=[
            jax.ShapeDtypeStruct((_B, _P), jnp.float32),
            jax.ShapeDtypeStruct((2 * _B, 160, 128), jnp.float32),
        ],
        scratch_shapes=[
            pltpu.VMEM((_B, _E), jnp.float32),
            pltpu.VMEM((_NBUF, _CH, 8, _E), jnp.float32),
            pltpu.SemaphoreType.DMA((_NBUF,)),
            pltpu.SemaphoreType.DMA((2 * _B,)),
        ],
    )(x4, W, b2, p3)
    e_prompt = ep.reshape(1, _B, 2, 5, 32, 128)
    return (logits, e_prompt)


# speedup vs baseline: 1.0111x; 1.0111x over previous
import jax
import jax.numpy as jnp
from jax.experimental import pallas as pl
from jax.experimental.pallas import tpu as pltpu

_B, _S, _E = 4, 2048, 4096
_P = 10
_R = _S // 8
_CH = 16
_NBUF = 6
_NCH = _R // _CH


def _copy(x_hbm, bufs, sems, c):
    bi, k = divmod(c, _NCH)
    return pltpu.make_async_copy(
        x_hbm.at[bi, pl.ds(k * _CH, _CH)],
        bufs.at[c % _NBUF],
        sems.at[c % _NBUF],
    )


def _body(x_hbm, w_ref, b_ref, p_hbm, logits_ref, ep_ref,
          accs, bufs, sems, gsems):
    total = _B * _NCH
    for c in range(_NBUF):
        _copy(x_hbm, bufs, sems, c).start()
    acc = None
    for c in range(total):
        bi, k = divmod(c, _NCH)
        _copy(x_hbm, bufs, sems, c).wait()
        part = jnp.max(bufs[c % _NBUF], axis=0)
        acc = part if k == 0 else jnp.maximum(acc, part)
        if c + _NBUF < total:
            _copy(x_hbm, bufs, sems, c + _NBUF).start()
        if k == _NCH - 1:
            accs[pl.ds(bi, 1), :] = jnp.max(acc, axis=0, keepdims=True)
    logits = jax.lax.dot_general(
        accs[...], w_ref[...],
        (((1,), (1,)), ((), ())),
        preferred_element_type=jnp.float32,
    ) + b_ref[...]
    logits_ref[...] = logits
    iota = jax.lax.broadcasted_iota(jnp.int32, (_B, _P), 1)
    m = jnp.max(logits, axis=1, keepdims=True)
    idxv = jnp.min(jnp.where(logits == m, iota, _P), axis=1, keepdims=True)
    gathers = []
    for bi in range(_B):
        rid = idxv[bi, 0]
        for t in range(2):
            g = pltpu.make_async_copy(
                p_hbm.at[rid + t * _P],
                ep_ref.at[2 * bi + t],
                gsems.at[2 * bi + t],
            )
            g.start()
            gathers.append(g)
    for g in gathers:
        g.wait()


def kernel(x_embed, prompt, W, b):
    x4 = x_embed.reshape(_B, _R, 8, _E)
    p3 = prompt.reshape(2 * _P, 160, 128)
    b2 = b.reshape(1, _P)
    logits, ep = pl.pallas_call(
        _body,
        in_specs=[
            pl.BlockSpec(memory_space=pltpu.MemorySpace.HBM),
            pl.BlockSpec(memory_space=pltpu.MemorySpace.VMEM),
            pl.BlockSpec(memory_space=pltpu.MemorySpace.VMEM),
            pl.BlockSpec(memory_space=pltpu.MemorySpace.HBM),
        ],
        out_specs=[
            pl.BlockSpec(memory_space=pltpu.MemorySpace.VMEM),
            pl.BlockSpec(memory_space=pltpu.MemorySpace.VMEM),
        ],
        out_shape=[
            jax.ShapeDtypeStruct((_B, _P), jnp.float32),
            jax.ShapeDtypeStruct((2 * _B, 160, 128), jnp.float32),
        ],
        scratch_shapes=[
            pltpu.VMEM((_B, _E), jnp.float32),
            pltpu.VMEM((_NBUF, _CH, 8, _E), jnp.float32),
            pltpu.SemaphoreType.DMA((_NBUF,)),
            pltpu.SemaphoreType.DMA((2 * _B,)),
        ],
    )(x4, W, b2, p3)
    e_prompt = ep.reshape(1, _B, 2, 5, 32, 128)
    return (logits, e_prompt)


# speedup vs baseline: 1.0320x; 1.0207x over previous
import jax
import jax.numpy as jnp
from jax.experimental import pallas as pl
from jax.experimental.pallas import tpu as pltpu

_B, _S, _E = 4, 2048, 4096
_P = 10
_R = _S // 8
_CH = 16
_NBUF = 4
_NCH = _R // _CH


def _copy(x_hbm, bufs, sems, c):
    bi, k = divmod(c, _NCH)
    return pltpu.make_async_copy(
        x_hbm.at[bi, pl.ds(k * _CH, _CH)],
        bufs.at[c % _NBUF],
        sems.at[c % _NBUF],
    )


def _body(x_hbm, w_ref, b_ref, p_hbm, logits_ref, ep_ref,
          accs, bufs, sems, gsems):
    total = _B * _NCH
    for c in range(_NBUF):
        _copy(x_hbm, bufs, sems, c).start()
    acc = None
    gathers = []
    for c in range(total):
        bi, k = divmod(c, _NCH)
        _copy(x_hbm, bufs, sems, c).wait()
        part = jnp.max(bufs[c % _NBUF], axis=0)
        acc = part if k == 0 else jnp.maximum(acc, part)
        if c + _NBUF < total:
            _copy(x_hbm, bufs, sems, c + _NBUF).start()
        if k == _NCH - 1:
            red = jnp.max(acc, axis=0, keepdims=True)
            logits = jax.lax.dot_general(
                red, w_ref[...],
                (((1,), (1,)), ((), ())),
                preferred_element_type=jnp.float32,
            ) + b_ref[...]
            logits_ref[pl.ds(bi, 1), :] = logits
            iota = jax.lax.broadcasted_iota(jnp.int32, (1, _P), 1)
            m = jnp.max(logits, axis=1, keepdims=True)
            idx = jnp.min(jnp.where(logits == m, iota, _P), axis=1,
                          keepdims=True)
            rid = idx[0, 0]
            for t in range(2):
                g = pltpu.make_async_copy(
                    p_hbm.at[rid + t * _P],
                    ep_ref.at[2 * bi + t],
                    gsems.at[2 * bi + t],
                )
                g.start()
                gathers.append(g)
    for g in gathers:
        g.wait()


def kernel(x_embed, prompt, W, b):
    x4 = x_embed.reshape(_B, _R, 8, _E)
    p3 = prompt.reshape(2 * _P, 160, 128)
    b2 = b.reshape(1, _P)
    logits, ep = pl.pallas_call(
        _body,
        in_specs=[
            pl.BlockSpec(memory_space=pltpu.MemorySpace.HBM),
            pl.BlockSpec(memory_space=pltpu.MemorySpace.VMEM),
            pl.BlockSpec(memory_space=pltpu.MemorySpace.VMEM),
            pl.BlockSpec(memory_space=pltpu.MemorySpace.HBM),
        ],
        out_specs=[
            pl.BlockSpec(memory_space=pltpu.MemorySpace.VMEM),
            pl.BlockSpec(memory_space=pltpu.MemorySpace.VMEM),
        ],
        out_shape=[
            jax.ShapeDtypeStruct((_B, _P), jnp.float32),
            jax.ShapeDtypeStruct((2 * _B, 160, 128), jnp.float32),
        ],
        scratch_shapes=[
            pltpu.VMEM((_B, _E), jnp.float32),
            pltpu.VMEM((_NBUF, _CH, 8, _E), jnp.float32),
            pltpu.SemaphoreType.DMA((_NBUF,)),
            pltpu.SemaphoreType.DMA((2 * _B,)),
        ],
    )(x4, W, b2, p3)
    e_prompt = ep.reshape(1, _B, 2, 5, 32, 128)
    return (logits, e_prompt)
